# baseline (speedup 1.0000x reference)
import functools

import jax
import jax.numpy as jnp
from jax import lax
from jax.experimental import pallas as pl
from jax.experimental.pallas import tpu as pltpu
from jax.experimental.pallas import tpu_sc as plsc

SBLK = 256


def _tc_body(nsteps, x_ref, w_ref, b_ref, logits_ref, idx_ref,
             mx_acc, idx_acc):
    k = pl.program_id(1)
    xb = x_ref[...]
    _, SB, D = xb.shape
    blk_mx = jnp.max(xb, axis=1)
    iota_s = lax.broadcasted_iota(jnp.int32, (1, SB, D), 1) + k * SB
    blk_idx = jnp.min(
        jnp.where(xb == blk_mx[:, None, :], iota_s, jnp.int32(nsteps * SB)),
        axis=1)

    @pl.when(k == 0)
    def _():
        mx_acc[...] = blk_mx
        idx_acc[...] = blk_idx

    @pl.when(k > 0)
    def _():
        better = blk_mx > mx_acc[...]
        mx_acc[...] = jnp.where(better, blk_mx, mx_acc[...])
        idx_acc[...] = jnp.where(better, blk_idx, idx_acc[...])

    @pl.when(k == nsteps - 1)
    def _():
        i = pl.program_id(0)
        idx_ref[pl.ds(i, 1), :] = idx_acc[...]
        logits_ref[pl.ds(i, 1), :] = lax.dot_general(
            mx_acc[...], w_ref[...], (((1,), (1,)), ((), ())),
            preferred_element_type=jnp.float32) + b_ref[...]


def _make_sc_hist(B, S, D, NC, NS, L):
    mesh = plsc.VectorSubcoreMesh(core_axis_name="c", subcore_axis_name="s")
    b_per_core = B // NC
    d_per_tile = D // NS
    bins_core = b_per_core * S
    bins_tile = bins_core // NS
    s_per_tile = S // bins_tile
    inv = 1.0 / D

    @functools.partial(
        pl.kernel, mesh=mesh,
        out_type=jax.ShapeDtypeStruct((B, S), jnp.float32),
        compiler_params=pltpu.CompilerParams(needs_layout_passes=False),
        scratch_types=[
            pltpu.VMEM((b_per_core * d_per_tile,), jnp.int32),
            pltpu.VMEM((bins_core,), jnp.float32),
            pltpu.VMEM_SHARED((NS * bins_core,), jnp.float32),
            pltpu.VMEM((NS * bins_tile,), jnp.float32),
            pltpu.VMEM((bins_tile,), jnp.float32),
        ],
    )
    def hist_kernel(idx_hbm, out_hbm, idx_v, hist_v, shared, red_v, out_v):
        cid = lax.axis_index("c")
        sid = lax.axis_index("s")
        zero = jnp.zeros((L,), jnp.float32)
        for i in range(bins_core // L):
            hist_v[pl.ds(i * L, L)] = zero
        for k in range(b_per_core):
            pltpu.sync_copy(
                idx_hbm.at[b_per_core * cid + k,
                           pl.ds(sid * d_per_tile, d_per_tile)],
                idx_v.at[pl.ds(k * d_per_tile, d_per_tile)])
        val = jnp.full((L,), inv, jnp.float32)
        for ch in range(b_per_core * d_per_tile // L):
            k = ch // (d_per_tile // L)
            offs = idx_v[pl.ds(ch * L, L)]
            if k:
                offs = offs + jnp.int32(k * S)
            plsc.addupdate_scatter(hist_v, [offs], val)
        pltpu.sync_copy(hist_v, shared.at[pl.ds(sid * bins_core, bins_core)])
        plsc.subcore_barrier()
        for t in range(NS):
            pltpu.sync_copy(
                shared.at[pl.ds(t * bins_core + sid * bins_tile, bins_tile)],
                red_v.at[pl.ds(t * bins_tile, bins_tile)])
        for ch in range(bins_tile // L):
            acc = red_v[pl.ds(ch * L, L)]
            for t in range(1, NS):
                acc = acc + red_v[pl.ds(t * bins_tile + ch * L, L)]
            out_v[pl.ds(ch * L, L)] = acc
        pltpu.sync_copy(
            out_v,
            out_hbm.at[b_per_core * cid + sid // s_per_tile,
                       pl.ds((sid % s_per_tile) * bins_tile, bins_tile)])

    return hist_kernel


def kernel(x, W, b):
    B, S, D = x.shape
    N = W.shape[0]
    b2 = b.reshape(1, N)
    K = S // SBLK

    logits, idx = pl.pallas_call(
        functools.partial(_tc_body, K),
        grid=(B, K),
        in_specs=[
            pl.BlockSpec((1, SBLK, D), lambda i, k: (i, k, 0)),
            pl.BlockSpec((N, D), lambda i, k: (0, 0)),
            pl.BlockSpec((1, N), lambda i, k: (0, 0)),
        ],
        out_specs=[
            pl.BlockSpec((B, N), lambda i, k: (0, 0)),
            pl.BlockSpec((B, D), lambda i, k: (0, 0)),
        ],
        out_shape=[
            jax.ShapeDtypeStruct((B, N), jnp.float32),
            jax.ShapeDtypeStruct((B, D), jnp.int32),
        ],
        scratch_shapes=[
            pltpu.VMEM((1, D), jnp.float32),
            pltpu.VMEM((1, D), jnp.int32),
        ],
    )(x, W, b2)

    info = plsc.get_sparse_core_info()
    sc_hist = _make_sc_hist(B, S, D, info.num_cores, info.num_subcores,
                            info.num_lanes)
    return logits, sc_hist(idx)


# speedup vs baseline: 1.3205x; 1.3205x over previous
import functools

import jax
import jax.numpy as jnp
from jax import lax
from jax.experimental import pallas as pl
from jax.experimental.pallas import tpu as pltpu
from jax.experimental.pallas import tpu_sc as plsc

DBLK = 512


def _tc_body(x_ref, w_ref, b_ref, logits_ref, idx_ref):
    j = pl.program_id(0)
    xb = x_ref[...]
    B, S, D = xb.shape
    mx = jnp.max(xb, axis=1)
    iota_s = lax.broadcasted_iota(jnp.int32, (B, S, D), 1)
    idx_ref[...] = jnp.min(
        jnp.where(xb == mx[:, None, :], iota_s, jnp.int32(S)), axis=1)
    part = lax.dot_general(mx, w_ref[...], (((1,), (1,)), ((), ())),
                           preferred_element_type=jnp.float32)

    @pl.when(j == 0)
    def _():
        logits_ref[...] = part + b_ref[...]

    @pl.when(j > 0)
    def _():
        logits_ref[...] += part


def _make_sc_hist(B, S, D, NC, NS, L):
    mesh = plsc.VectorSubcoreMesh(core_axis_name="c", subcore_axis_name="s")
    b_per_core = B // NC
    d_per_tile = D // NS
    bins_core = b_per_core * S
    bins_tile = bins_core // NS
    s_per_tile = S // bins_tile
    inv = 1.0 / D

    @functools.partial(
        pl.kernel, mesh=mesh,
        out_type=jax.ShapeDtypeStruct((B, S), jnp.float32),
        compiler_params=pltpu.CompilerParams(needs_layout_passes=False),
        scratch_types=[
            pltpu.VMEM((b_per_core * d_per_tile,), jnp.int32),
            pltpu.VMEM((bins_core,), jnp.float32),
            pltpu.VMEM_SHARED((NS, bins_core), jnp.float32),
            pltpu.VMEM((NS, bins_tile), jnp.float32),
            pltpu.VMEM((bins_tile,), jnp.float32),
        ],
    )
    def hist_kernel(idx_hbm, out_hbm, idx_v, hist_v, shared, red_v, out_v):
        cid = lax.axis_index("c")
        sid = lax.axis_index("s")
        zero = jnp.zeros((L,), jnp.float32)
        for i in range(bins_core // L):
            hist_v[pl.ds(i * L, L)] = zero
        for k in range(b_per_core):
            pltpu.sync_copy(
                idx_hbm.at[b_per_core * cid + k,
                           pl.ds(sid * d_per_tile, d_per_tile)],
                idx_v.at[pl.ds(k * d_per_tile, d_per_tile)])
        val = jnp.full((L,), inv, jnp.float32)
        for ch in range(b_per_core * d_per_tile // L):
            k = ch // (d_per_tile // L)
            offs = idx_v[pl.ds(ch * L, L)]
            if k:
                offs = offs + jnp.int32(k * S)
            plsc.addupdate_scatter(hist_v, [offs], val)
        pltpu.sync_copy(hist_v, shared.at[sid])
        plsc.subcore_barrier()
        pltpu.sync_copy(shared.at[:, pl.ds(sid * bins_tile, bins_tile)],
                        red_v)
        for ch in range(bins_tile // L):
            acc = red_v[0, pl.ds(ch * L, L)]
            for t in range(1, NS):
                acc = acc + red_v[t, pl.ds(ch * L, L)]
            out_v[pl.ds(ch * L, L)] = acc
        pltpu.sync_copy(
            out_v,
            out_hbm.at[b_per_core * cid + sid // s_per_tile,
                       pl.ds((sid % s_per_tile) * bins_tile, bins_tile)])

    return hist_kernel


def kernel(x, W, b):
    B, S, D = x.shape
    N = W.shape[0]
    b2 = b.reshape(1, N)
    G = D // DBLK

    logits, idx = pl.pallas_call(
        _tc_body,
        grid=(G,),
        in_specs=[
            pl.BlockSpec((B, S, DBLK), lambda j: (0, 0, j)),
            pl.BlockSpec((N, DBLK), lambda j: (0, j)),
            pl.BlockSpec((1, N), lambda j: (0, 0)),
        ],
        out_specs=[
            pl.BlockSpec((B, N), lambda j: (0, 0)),
            pl.BlockSpec((B, DBLK), lambda j: (0, j)),
        ],
        out_shape=[
            jax.ShapeDtypeStruct((B, N), jnp.float32),
            jax.ShapeDtypeStruct((B, D), jnp.int32),
        ],
    )(x, W, b2)

    info = plsc.get_sparse_core_info()
    sc_hist = _make_sc_hist(B, S, D, info.num_cores, info.num_subcores,
                            info.num_lanes)
    return logits, sc_hist(idx)
